# baseline (speedup 1.0000x reference)
import jax
import jax.numpy as jnp
from jax.experimental import pallas as pl
from jax.experimental.pallas import tpu as pltpu

_T = 32


def _copy_body(in_ref, out_ref):
    out_ref[...] = in_ref[...]


def _copy3d(x, block):
    n, t, width = x.shape
    grid = n // block
    return pl.pallas_call(
        _copy_body,
        out_shape=jax.ShapeDtypeStruct(x.shape, x.dtype),
        grid=(grid,),
        in_specs=[pl.BlockSpec((block, t, width), lambda i: (i, 0, 0))],
        out_specs=pl.BlockSpec((block, t, width), lambda i: (i, 0, 0)),
    )(x)


def kernel(emb_user, emb_item):
    u_shape, i_shape = emb_user.shape, emb_item.shape
    u3 = emb_user.reshape(-1, _T, 64)
    i3 = emb_item.reshape(-1, _T, 64)
    out_u = _copy3d(u3, 250)
    out_i = _copy3d(i3, 625)
    return (out_u.reshape(u_shape), out_i.reshape(i_shape))
